# Initial kernel scaffold; baseline (speedup 1.0000x reference)
import jax
import jax.numpy as jnp
from jax.experimental import pallas as pl


def kernel(clinical_numerical_input, clinical_categorical_input, tables, W1, b1, W2, b2):
    raise NotImplementedError("write your pallas kernel here")


# speedup vs baseline: 8.1868x; 8.1868x over previous
import functools

import jax
import jax.numpy as jnp
from jax import lax
from jax.experimental import pallas as pl
from jax.experimental.pallas import tpu as pltpu
from jax.experimental.pallas import tpu_sc as plsc

B = 16384
N_NUM = 13
N_CAT = 26
VOCAB = 100000
EMB = 16
LANES = 16

NC = 2
NS = 16
NW = NC * NS
RPW = B // NW
CHUNK = 64
NCHUNK = RPW // CHUNK
IPC = CHUNK * N_CAT
IDXW = 128
IGROUPS = IPC // IDXW
IDX_ROWS_PER_W = RPW * N_CAT // IDXW
DW = N_NUM + 3


def _sc_body(tab_hbm, idx_hbm, num_hbm, weff_hbm, out_hbm,
             idx_v, rows_v, num_v, weff_v, out_v, sem):
    cid = lax.axis_index("c")
    sid = lax.axis_index("s")
    wid = sid * NC + cid

    pltpu.sync_copy(weff_hbm, weff_v)
    pltpu.sync_copy(num_hbm.at[pl.ds(wid * RPW * LANES, RPW * LANES)], num_v)
    pltpu.sync_copy(idx_hbm.at[pl.ds(wid * IDX_ROWS_PER_W, IDX_ROWS_PER_W), :], idx_v)
    w_num = weff_v[pl.ds(0, LANES)]
    iota = lax.iota(jnp.int32, LANES)
    col_base = iota * LANES

    def chunk_body(g, carry):
        copies = [
            pltpu.async_copy(
                tab_hbm.at[idx_v.at[g * IGROUPS + j]],
                rows_v.at[pl.ds(j * IDXW, IDXW), :],
                sem,
            )
            for j in range(IGROUPS)
        ]
        for cp in copies:
            cp.wait()

        def group_body(t, c2):
            r0 = t * LANES
            out_vec = jnp.zeros((LANES,), jnp.float32)
            for jj in range(LANES):
                r = r0 + jj
                acc = num_v[pl.ds((g * CHUNK + r) * LANES, LANES)] * w_num
                for c in range(N_CAT):
                    acc = acc + rows_v[r * N_CAT + c, :] * weff_v[pl.ds(LANES + c * LANES, LANES)]
                tot = jnp.broadcast_to(jnp.sum(acc), (LANES,))
                out_vec = jnp.where(iota == jj, tot, out_vec)
            out_v[pl.ds(g * CHUNK + r0, LANES)] = out_vec
            return c2

        lax.fori_loop(0, CHUNK // LANES, group_body, 0)
        return carry

    lax.fori_loop(0, NCHUNK, chunk_body, 0)
    pltpu.sync_copy(out_v, out_hbm.at[pl.ds(wid * RPW, RPW)])


_sc_call = functools.partial(
    pl.kernel,
    mesh=plsc.VectorSubcoreMesh(core_axis_name="c", subcore_axis_name="s"),
    out_type=jax.ShapeDtypeStruct((B,), jnp.float32),
    compiler_params=pltpu.CompilerParams(
        needs_layout_passes=False, use_tc_tiling_on_sc=False),
    scratch_types=[
        pltpu.VMEM((IDX_ROWS_PER_W, IDXW), jnp.int32),
        pltpu.VMEM((IPC, EMB), jnp.float32),
        pltpu.VMEM((RPW * LANES,), jnp.float32),
        pltpu.VMEM(((1 + N_CAT) * LANES,), jnp.float32),
        pltpu.VMEM((RPW,), jnp.float32),
        pltpu.SemaphoreType.DMA,
    ],
)(_sc_body)


def kernel(clinical_numerical_input, clinical_categorical_input, tables, W1, b1, W2, b2):
    weff = (W1 @ W2)[:, 0]
    beff = (b1 @ W2 + b2)[0]
    weff_vec = jnp.concatenate([
        weff[:N_NUM],
        beff[None],
        jnp.zeros((2,), jnp.float32),
        weff[N_NUM:],
    ])
    num_pad = jnp.concatenate([
        clinical_numerical_input,
        jnp.ones((B, 1), jnp.float32),
        jnp.zeros((B, 2), jnp.float32),
    ], axis=1).reshape(B * LANES)
    idx = (clinical_categorical_input
           + (jnp.arange(N_CAT, dtype=jnp.int32) * VOCAB)[None, :])
    idx2d = idx.reshape(B * N_CAT // IDXW, IDXW)
    tab_flat = tables.reshape(N_CAT * VOCAB, EMB)
    out = _sc_call(tab_flat, idx2d, num_pad, weff_vec)
    return out.reshape(B, 1)


# speedup vs baseline: 33.8507x; 4.1348x over previous
import functools

import jax
import jax.numpy as jnp
from jax import lax
from jax.experimental import pallas as pl
from jax.experimental.pallas import tpu as pltpu
from jax.experimental.pallas import tpu_sc as plsc

B = 16384
N_NUM = 13
N_CAT = 26
VOCAB = 100000
EMB = 16
LANES = 16

NC = 2
NS = 16
NW = NC * NS
RPW = B // NW
IDXW = 128
IDX_ROWS_PER_W = RPW * N_CAT // IDXW
SPW = RPW * N_CAT
VCH = 10000

_MESH = plsc.VectorSubcoreMesh(core_axis_name="c", subcore_axis_name="s")
_PARAMS = pltpu.CompilerParams(
    needs_layout_passes=False, use_tc_tiling_on_sc=False)


def _scores_tc_body(tabt_ref, w_ref, out_ref):
    c = pl.program_id(0)
    blk = tabt_ref[0]
    w = w_ref[c]
    out_ref[...] = jnp.sum(blk * w[:, None], axis=0)[None, None, :]


def _scores_tc(tabt, wcat):
    return pl.pallas_call(
        _scores_tc_body,
        grid=(N_CAT,),
        in_specs=[
            pl.BlockSpec((1, EMB, VOCAB), lambda c: (c, 0, 0)),
            pl.BlockSpec((N_CAT, EMB), lambda c: (0, 0)),
        ],
        out_specs=pl.BlockSpec((1, 1, VOCAB), lambda c: (c, 0, 0)),
        out_shape=jax.ShapeDtypeStruct((N_CAT, 1, VOCAB), jnp.float32),
    )(tabt, wcat)


def _gather_body(scores_hbm, idx_hbm, num_hbm, wnum_hbm, out_hbm,
                 idx_v, sv_v, num_v, wnum_v, out_v, sem):
    wid = lax.axis_index("s") * NC + lax.axis_index("c")
    pltpu.sync_copy(wnum_hbm, wnum_v)
    pltpu.sync_copy(num_hbm.at[pl.ds(wid * RPW * LANES, RPW * LANES)], num_v)
    pltpu.sync_copy(idx_hbm.at[pl.ds(wid * IDX_ROWS_PER_W, IDX_ROWS_PER_W), :],
                    idx_v)
    w_num = wnum_v[pl.ds(0, LANES)]
    iota = lax.iota(jnp.int32, LANES)
    mask10 = iota < 10
    zeros = jnp.zeros((LANES,), jnp.float32)

    def gather8(j8, carry):
        copies = [
            pltpu.async_copy(
                scores_hbm.at[idx_v.at[j8 * 8 + j]],
                sv_v.at[pl.ds((j8 * 8 + j) * IDXW, IDXW)],
                sem,
            )
            for j in range(8)
        ]
        for cp in copies:
            cp.wait()
        return carry

    lax.fori_loop(0, IDX_ROWS_PER_W // 8, gather8, 0)

    def group_body(t, c2):
        r0 = t * LANES
        out_vec = zeros
        for jj in range(LANES):
            r = r0 + jj
            base = r * N_CAT
            s1 = sv_v[pl.ds(base, LANES)]
            s2 = sv_v[pl.ds(base + LANES, LANES)]
            acc = (s1 + jnp.where(mask10, s2, zeros)
                   + num_v[pl.ds(r * LANES, LANES)] * w_num)
            tot = jnp.broadcast_to(jnp.sum(acc), (LANES,))
            out_vec = jnp.where(iota == jj, tot, out_vec)
        out_v[pl.ds(r0, LANES)] = out_vec
        return c2

    lax.fori_loop(0, RPW // LANES, group_body, 0)
    pltpu.sync_copy(out_v, out_hbm.at[pl.ds(wid * RPW, RPW)])


_gather_call = functools.partial(
    pl.kernel,
    mesh=_MESH,
    out_type=jax.ShapeDtypeStruct((B,), jnp.float32),
    scratch_types=[
        pltpu.VMEM((IDX_ROWS_PER_W, IDXW), jnp.int32),
        pltpu.VMEM((SPW + 2 * LANES,), jnp.float32),
        pltpu.VMEM((RPW * LANES,), jnp.float32),
        pltpu.VMEM((LANES,), jnp.float32),
        pltpu.VMEM((RPW,), jnp.float32),
        pltpu.SemaphoreType.DMA,
    ],
    compiler_params=_PARAMS,
)(_gather_body)


def kernel(clinical_numerical_input, clinical_categorical_input, tables, W1, b1, W2, b2):
    weff = (W1 @ W2)[:, 0]
    beff = (b1 @ W2 + b2)[0]
    wcat = weff[N_NUM:].reshape(N_CAT, EMB)
    wnum = jnp.concatenate([
        weff[:N_NUM], beff[None], jnp.zeros((2,), jnp.float32)])
    tabt = jnp.transpose(tables, (0, 2, 1))
    scores = _scores_tc(tabt, wcat)
    num_pad = jnp.concatenate([
        clinical_numerical_input,
        jnp.ones((B, 1), jnp.float32),
        jnp.zeros((B, 2), jnp.float32),
    ], axis=1).reshape(B * LANES)
    idx = (clinical_categorical_input
           + (jnp.arange(N_CAT, dtype=jnp.int32) * VOCAB)[None, :])
    idx2d = idx.reshape(B * N_CAT // IDXW, IDXW)
    out = _gather_call(scores.reshape(N_CAT * VOCAB), idx2d, num_pad, wnum)
    return out.reshape(B, 1)


# speedup vs baseline: 64.0676x; 1.8927x over previous
import functools

import jax
import jax.numpy as jnp
from jax import lax
from jax.experimental import pallas as pl
from jax.experimental.pallas import tpu as pltpu
from jax.experimental.pallas import tpu_sc as plsc

B = 16384
N_NUM = 13
N_CAT = 26
VOCAB = 100000
EMB = 16
LANES = 16

NC = 2
NS = 16
NW = NC * NS
RPW = B // NW
IDXW = 128
IDX_ROWS_PER_W = RPW * N_CAT // IDXW
SPW = RPW * N_CAT
VPAD = 100352

_MESH = plsc.VectorSubcoreMesh(core_axis_name="c", subcore_axis_name="s")
_PARAMS = pltpu.CompilerParams(
    needs_layout_passes=False, use_tc_tiling_on_sc=False)


def _scores_tc_body(tabt_ref, w_ref, out_ref):
    c = pl.program_id(0)
    blk = tabt_ref[0]
    w = w_ref[c]
    s = jnp.sum(blk * w[:, None], axis=0)
    out_ref[...] = jnp.concatenate(
        [s, jnp.zeros((VPAD - VOCAB,), jnp.float32)])


def _scores_tc(tabt, wcat):
    return pl.pallas_call(
        _scores_tc_body,
        grid=(N_CAT,),
        in_specs=[
            pl.BlockSpec((1, EMB, VOCAB), lambda c: (c, 0, 0)),
            pl.BlockSpec((N_CAT, EMB), lambda c: (0, 0)),
        ],
        out_specs=pl.BlockSpec((VPAD,), lambda c: (c,)),
        out_shape=jax.ShapeDtypeStruct((N_CAT * VPAD,), jnp.float32),
    )(tabt, wcat)


def _gather_body(scores_hbm, idx_hbm, num_hbm, wnum_hbm, out_hbm,
                 idx_v, sv_v, num_v, wnum_v, out_v, sem):
    wid = lax.axis_index("s") * NC + lax.axis_index("c")
    pltpu.sync_copy(wnum_hbm, wnum_v)
    pltpu.sync_copy(num_hbm.at[pl.ds(wid * RPW * LANES, RPW * LANES)], num_v)
    pltpu.sync_copy(idx_hbm.at[pl.ds(wid * IDX_ROWS_PER_W, IDX_ROWS_PER_W), :],
                    idx_v)
    w_num = wnum_v[pl.ds(0, LANES)]
    iota = lax.iota(jnp.int32, LANES)
    mask10 = iota < 10
    zeros = jnp.zeros((LANES,), jnp.float32)

    def gather8(j8, carry):
        copies = [
            pltpu.async_copy(
                scores_hbm.at[idx_v.at[j8 * 8 + j]],
                sv_v.at[pl.ds((j8 * 8 + j) * IDXW, IDXW)],
                sem,
            )
            for j in range(8)
        ]
        for cp in copies:
            cp.wait()
        return carry

    lax.fori_loop(0, IDX_ROWS_PER_W // 8, gather8, 0)

    def group_body(t, c2):
        r0 = t * LANES
        out_vec = zeros
        for jj in range(LANES):
            r = r0 + jj
            base = r * N_CAT
            s1 = sv_v[pl.ds(base, LANES)]
            s2 = sv_v[pl.ds(base + LANES, LANES)]
            acc = (s1 + jnp.where(mask10, s2, zeros)
                   + num_v[pl.ds(r * LANES, LANES)] * w_num)
            tot = jnp.broadcast_to(jnp.sum(acc), (LANES,))
            out_vec = jnp.where(iota == jj, tot, out_vec)
        out_v[pl.ds(r0, LANES)] = out_vec
        return c2

    lax.fori_loop(0, RPW // LANES, group_body, 0)
    pltpu.sync_copy(out_v, out_hbm.at[pl.ds(wid * RPW, RPW)])


_gather_call = functools.partial(
    pl.kernel,
    mesh=_MESH,
    out_type=jax.ShapeDtypeStruct((B,), jnp.float32),
    scratch_types=[
        pltpu.VMEM((IDX_ROWS_PER_W, IDXW), jnp.int32),
        pltpu.VMEM((SPW + 2 * LANES,), jnp.float32),
        pltpu.VMEM((RPW * LANES,), jnp.float32),
        pltpu.VMEM((LANES,), jnp.float32),
        pltpu.VMEM((RPW,), jnp.float32),
        pltpu.SemaphoreType.DMA,
    ],
    compiler_params=_PARAMS,
)(_gather_body)


def kernel(clinical_numerical_input, clinical_categorical_input, tables, W1, b1, W2, b2):
    weff = (W1 @ W2)[:, 0]
    beff = (b1 @ W2 + b2)[0]
    wcat = weff[N_NUM:].reshape(N_CAT, EMB)
    wnum = jnp.concatenate([
        weff[:N_NUM], beff[None], jnp.zeros((2,), jnp.float32)])
    tabt = jnp.transpose(tables, (0, 2, 1))
    scores = _scores_tc(tabt, wcat)
    num_pad = jnp.concatenate([
        clinical_numerical_input,
        jnp.ones((B, 1), jnp.float32),
        jnp.zeros((B, 2), jnp.float32),
    ], axis=1).reshape(B * LANES)
    idx = (clinical_categorical_input
           + (jnp.arange(N_CAT, dtype=jnp.int32) * VPAD)[None, :])
    idx2d = idx.reshape(B * N_CAT // IDXW, IDXW)
    out = _gather_call(scores, idx2d, num_pad, wnum)
    return out.reshape(B, 1)
